# baseline (speedup 1.0000x reference)
import functools

import jax
import jax.numpy as jnp
from jax import lax
from jax.experimental import pallas as pl
from jax.experimental.pallas import tpu as pltpu
from jax.experimental.pallas import tpu_sc as plsc

N_ENT = 100000
N_REL = 1000
D = 64
PW = 32
W = 128
B = 16384
MULT = 20.0

NC = 2
NS = 16
L = 16
NW = NC * NS
BPW = B // NW
CH = 64
NCHUNK = BPW // CH
NG = CH // L

EBLK = 8192
HB = EBLK // 2
NEB = pl.cdiv(N_ENT, EBLK)
NROW = NEB * HB


def _ident():
    r = lax.broadcasted_iota(jnp.int32, (D, D), 0)
    c = lax.broadcasted_iota(jnp.int32, (D, D), 1)
    return jnp.where(r == c, 1.0, 0.0).astype(jnp.float32)


_DN_T = (((0,), (0,)), ((), ()))


def _mxu_t(x, ident):
    return lax.dot_general(x, ident, _DN_T, preferred_element_type=jnp.float32)


def _pack(t):
    ti = lax.bitcast_convert_type(t, jnp.int32)
    r = (ti + 0x7FFF + ((ti >> 16) & 1)) >> 16
    return (r[:, 0:PW] & 0xFFFF) | (r[:, PW:D] << 16)


def _prep_body(et_ref, ett_ref, rt_ref, rhtt_ref, rttt_ref,
               out_ref, rri_ref):
    ident = _ident()
    pe = _pack(_mxu_t(et_ref[...], ident))
    pt = _pack(_mxu_t(ett_ref[...], ident))
    out_ref[:, 0:PW] = pe[0:HB]
    out_ref[:, PW:2 * PW] = pt[0:HB]
    out_ref[:, 2 * PW:3 * PW] = pe[HB:EBLK]
    out_ref[:, 3 * PW:W] = pt[HB:EBLK]

    @pl.when(pl.program_id(0) == 0)
    def _():
        pr = _pack(_mxu_t(rt_ref[...], ident))
        ph = _pack(_mxu_t(rhtt_ref[...], ident))
        ptt = _pack(_mxu_t(rttt_ref[...], ident))
        rri_ref[:, 0:PW] = pr
        rri_ref[:, PW:2 * PW] = ph
        rri_ref[:, 2 * PW:3 * PW] = ptt
        rri_ref[:, 3 * PW:W] = jnp.zeros_like(pr)


_prep = pl.pallas_call(
    _prep_body,
    compiler_params=pltpu.CompilerParams(fuse_transposed_lhs_in_matmul=True),
    grid=(NEB,),
    in_specs=[
        pl.BlockSpec((D, EBLK), lambda i: (0, i)),
        pl.BlockSpec((D, EBLK), lambda i: (0, i)),
        pl.BlockSpec((D, N_REL), lambda i: (0, 0)),
        pl.BlockSpec((D, N_REL), lambda i: (0, 0)),
        pl.BlockSpec((D, N_REL), lambda i: (0, 0)),
    ],
    out_specs=[
        pl.BlockSpec((HB, W), lambda i: (i, 0)),
        pl.BlockSpec((N_REL, W), lambda i: (0, 0)),
    ],
    out_shape=[
        jax.ShapeDtypeStruct((NROW, W), jnp.int32),
        jax.ShapeDtypeStruct((N_REL, W), jnp.int32),
    ],
)

_mesh = plsc.VectorSubcoreMesh(core_axis_name="c", subcore_axis_name="s")


@functools.partial(
    pl.kernel,
    out_type=jax.ShapeDtypeStruct((B,), jnp.float32),
    mesh=_mesh,
    compiler_params=pltpu.CompilerParams(
        needs_layout_passes=False, use_tc_tiling_on_sc=True),
    scratch_types=[
        pltpu.VMEM((BPW,), jnp.int32),
        pltpu.VMEM((BPW,), jnp.int32),
        pltpu.VMEM((BPW,), jnp.int32),
        pltpu.VMEM((BPW,), jnp.int32),
        pltpu.VMEM((BPW,), jnp.int32),
        pltpu.VMEM((CH, W), jnp.int32),
        pltpu.VMEM((CH, W), jnp.int32),
        pltpu.VMEM((CH, W), jnp.int32),
        pltpu.VMEM((CH, W), jnp.int32),
        pltpu.VMEM((CH, W), jnp.int32),
        pltpu.VMEM((CH, W), jnp.int32),
        pltpu.VMEM((BPW,), jnp.float32),
        pltpu.SemaphoreType.DMA,
        pltpu.SemaphoreType.DMA,
    ],
)
def _sc_score(s_hbm, r_hbm, o_hbm, eei_hbm, rri_hbm,
              out_hbm,
              sidx, ridx, oidx, sidx2, oidx2,
              srow0, orow0, rrow0,
              srow1, orow1, rrow1,
              outv, sem0, sem1):
    wid = lax.axis_index("s") * NC + lax.axis_index("c")
    base = pl.multiple_of(wid * BPW, BPW)

    sets = ((srow0, orow0, rrow0, sem0),
            (srow1, orow1, rrow1, sem1))

    def fire(c, bset):
        srow, orow, rrow, sem = bset
        off = pl.multiple_of(c * CH, CH)
        pltpu.async_copy(eei_hbm.at[sidx2.at[pl.ds(off, CH)]], srow, sem)
        pltpu.async_copy(eei_hbm.at[oidx2.at[pl.ds(off, CH)]], orow, sem)
        pltpu.async_copy(rri_hbm.at[ridx.at[pl.ds(off, CH)]], rrow, sem)

    def drain(c, bset):
        srow, orow, rrow, sem = bset
        off = pl.multiple_of(c * CH, CH)
        pltpu.make_async_copy(eei_hbm.at[sidx2.at[pl.ds(off, CH)]], srow, sem).wait()
        pltpu.make_async_copy(eei_hbm.at[oidx2.at[pl.ds(off, CH)]], orow, sem).wait()
        pltpu.make_async_copy(rri_hbm.at[ridx.at[pl.ds(off, CH)]], rrow, sem).wait()

    pltpu.sync_copy(s_hbm.at[pl.ds(base, BPW)], sidx)
    pltpu.sync_copy(r_hbm.at[pl.ds(base, BPW)], ridx)
    pltpu.sync_copy(o_hbm.at[pl.ds(base, BPW)], oidx)

    def rowify(k, _):
        sl = pl.ds(k * L, L)
        sv = sidx[sl]
        sidx2[sl] = ((sv >> 13) << 12) + (sv & (HB - 1))
        ov = oidx[sl]
        oidx2[sl] = ((ov >> 13) << 12) + (ov & (HB - 1))
        return _

    lax.fori_loop(0, BPW // L, rowify, 0)
    fire(0, sets[0])

    lane = lax.iota(jnp.int32, 16)

    def compute(c, bset):
        srow, orow, rrow, _ = bset
        for g in range(NG):
            tvec = lane + g * L
            gb = pl.ds(c * CH + g * L, L)
            s_woff = ((sidx[gb] >> 12) & 1) * D
            o_woff = ((oidx[gb] >> 12) & 1) * D

            def word_body(w, accs):
                a0, a1, a2, a3, a4, a5 = accs
                wv = (lane + w) & (PW - 1)
                s_wv = s_woff + wv
                o_wv = o_woff + wv
                s_e = plsc.load_gather(srow, [tvec, s_wv])
                s_t = plsc.load_gather(srow, [tvec, s_wv + PW])
                o_e = plsc.load_gather(orow, [tvec, o_wv])
                o_t = plsc.load_gather(orow, [tvec, o_wv + PW])
                r_e = plsc.load_gather(rrow, [tvec, wv])
                r_h = plsc.load_gather(rrow, [tvec, wv + PW])
                r_t = plsc.load_gather(rrow, [tvec, wv + 2 * PW])
                fmt = plsc.PackFormat.INTERLEAVED
                se_a, se_b = plsc.unpack(plsc.bitcast(s_e, jnp.bfloat16), format=fmt)
                st_a, st_b = plsc.unpack(plsc.bitcast(s_t, jnp.bfloat16), format=fmt)
                oe_a, oe_b = plsc.unpack(plsc.bitcast(o_e, jnp.bfloat16), format=fmt)
                ot_a, ot_b = plsc.unpack(plsc.bitcast(o_t, jnp.bfloat16), format=fmt)
                re_a, re_b = plsc.unpack(plsc.bitcast(r_e, jnp.bfloat16), format=fmt)
                rh_a, rh_b = plsc.unpack(plsc.bitcast(r_h, jnp.bfloat16), format=fmt)
                rt_a, rt_b = plsc.unpack(plsc.bitcast(r_t, jnp.bfloat16), format=fmt)
                a0 = a0 + se_a * re_a * oe_a
                a3 = a3 + se_b * re_b * oe_b
                a1 = a1 + st_a * rh_a
                a4 = a4 + st_b * rh_b
                a2 = a2 + ot_a * rt_a
                a5 = a5 + ot_b * rt_b
                return (a0, a1, a2, a3, a4, a5)

            z = jnp.zeros((L,), jnp.float32)
            acc = lax.fori_loop(0, PW, word_body, (z,) * 6)
            b_acc = acc[0] + acc[3]
            h_acc = acc[1] + acc[4]
            t_acc = acc[2] + acc[5]
            res = (MULT
                   / (1.0 + jnp.exp(-b_acc))
                   / (1.0 + jnp.exp(-h_acc))
                   / (1.0 + jnp.exp(-t_acc)))
            outv[gb] = res

    def pair_body(p, carry):
        for b in (0, 1):
            c = 2 * p + b

            @pl.when(c + 1 < NCHUNK)
            def _():
                fire(c + 1, sets[1 - b])

            drain(c, sets[b])
            compute(c, sets[b])
        return carry

    lax.fori_loop(0, NCHUNK // 2, pair_body, 0)
    pltpu.sync_copy(outv, out_hbm.at[pl.ds(base, BPW)])


def kernel(s, r, o, E, R, E_t, R_ht, R_tt):
    eei, rri = _prep(E.T, E_t.T, R.T, R_ht.T, R_tt.T)
    return _sc_score(s, r, o, eei, rri)


# speedup vs baseline: 1.2577x; 1.2577x over previous
import functools

import jax
import jax.numpy as jnp
from jax import lax
from jax.experimental import pallas as pl
from jax.experimental.pallas import tpu as pltpu
from jax.experimental.pallas import tpu_sc as plsc

N_ENT = 100000
N_REL = 1000
D = 64
W = 128
B = 16384
MULT = 20.0

NC = 2
NS = 16
L = 16
NW = NC * NS
BPW = B // NW
CH = 64
NCHUNK = BPW // CH
NG = CH // L

EBLK = 12544


def _ident():
    r = lax.broadcasted_iota(jnp.int32, (D, D), 0)
    c = lax.broadcasted_iota(jnp.int32, (D, D), 1)
    return jnp.where(r == c, 1.0, 0.0).astype(jnp.float32)


_DN_T = (((0,), (0,)), ((), ()))


def _mxu_t(x, ident):
    return lax.dot_general(x, ident, _DN_T, preferred_element_type=jnp.float32)


def _prep_body(et_ref, ett_ref, rt_ref, rhtt_ref, rttt_ref,
               out_ref, rp_ref, rhtt_out_ref):
    ident = _ident()
    out_ref[:, 0:D] = _mxu_t(et_ref[...], ident)
    out_ref[:, D:W] = _mxu_t(ett_ref[...], ident)

    @pl.when(pl.program_id(0) == 0)
    def _():
        r = _mxu_t(rt_ref[...], ident)
        rp_ref[:, 0:D] = r
        rp_ref[:, D:W] = jnp.zeros_like(r)
        rhtt_out_ref[:, 0:D] = _mxu_t(rhtt_ref[...], ident)
        rhtt_out_ref[:, D:W] = _mxu_t(rttt_ref[...], ident)


_prep = pl.pallas_call(
    _prep_body,
    compiler_params=pltpu.CompilerParams(fuse_transposed_lhs_in_matmul=True),
    grid=(pl.cdiv(N_ENT, EBLK),),
    in_specs=[
        pl.BlockSpec((D, EBLK), lambda i: (0, i)),
        pl.BlockSpec((D, EBLK), lambda i: (0, i)),
        pl.BlockSpec((D, N_REL), lambda i: (0, 0)),
        pl.BlockSpec((D, N_REL), lambda i: (0, 0)),
        pl.BlockSpec((D, N_REL), lambda i: (0, 0)),
    ],
    out_specs=[
        pl.BlockSpec((EBLK, W), lambda i: (i, 0)),
        pl.BlockSpec((N_REL, W), lambda i: (0, 0)),
        pl.BlockSpec((N_REL, W), lambda i: (0, 0)),
    ],
    out_shape=[
        jax.ShapeDtypeStruct((N_ENT, W), jnp.float32),
        jax.ShapeDtypeStruct((N_REL, W), jnp.float32),
        jax.ShapeDtypeStruct((N_REL, W), jnp.float32),
    ],
)

_mesh = plsc.VectorSubcoreMesh(core_axis_name="c", subcore_axis_name="s")


@functools.partial(
    pl.kernel,
    out_type=jax.ShapeDtypeStruct((B,), jnp.float32),
    mesh=_mesh,
    compiler_params=pltpu.CompilerParams(
        needs_layout_passes=False, use_tc_tiling_on_sc=True),
    scratch_types=[
        pltpu.VMEM((BPW,), jnp.int32),
        pltpu.VMEM((BPW,), jnp.int32),
        pltpu.VMEM((BPW,), jnp.int32),
        pltpu.VMEM((CH, W), jnp.float32),
        pltpu.VMEM((CH, W), jnp.float32),
        pltpu.VMEM((CH, W), jnp.float32),
        pltpu.VMEM((CH, W), jnp.float32),
        pltpu.VMEM((CH, W), jnp.float32),
        pltpu.VMEM((CH, W), jnp.float32),
        pltpu.VMEM((CH, W), jnp.float32),
        pltpu.VMEM((CH, W), jnp.float32),
        pltpu.VMEM((BPW,), jnp.float32),
        pltpu.SemaphoreType.DMA,
        pltpu.SemaphoreType.DMA,
    ],
)
def _sc_score(s_hbm, r_hbm, o_hbm, ee_hbm, rp_hbm, rhtt_hbm,
              out_hbm,
              sidx, ridx, oidx,
              srow0, orow0, rrow0, rtrow0,
              srow1, orow1, rrow1, rtrow1,
              outv, sem0, sem1):
    wid = lax.axis_index("s") * NC + lax.axis_index("c")
    base = pl.multiple_of(wid * BPW, BPW)

    sets = ((srow0, orow0, rrow0, rtrow0, sem0),
            (srow1, orow1, rrow1, rtrow1, sem1))

    def fire(c, bset):
        srow, orow, rrow, rtrow, sem = bset
        off = pl.multiple_of(c * CH, CH)
        pltpu.async_copy(ee_hbm.at[sidx.at[pl.ds(off, CH)]], srow, sem)
        pltpu.async_copy(ee_hbm.at[oidx.at[pl.ds(off, CH)]], orow, sem)
        pltpu.async_copy(rp_hbm.at[ridx.at[pl.ds(off, CH)]], rrow, sem)
        pltpu.async_copy(rhtt_hbm.at[ridx.at[pl.ds(off, CH)]], rtrow, sem)

    def drain(c, bset):
        srow, orow, rrow, rtrow, sem = bset
        off = pl.multiple_of(c * CH, CH)
        pltpu.make_async_copy(ee_hbm.at[sidx.at[pl.ds(off, CH)]], srow, sem).wait()
        pltpu.make_async_copy(ee_hbm.at[oidx.at[pl.ds(off, CH)]], orow, sem).wait()
        pltpu.make_async_copy(rp_hbm.at[ridx.at[pl.ds(off, CH)]], rrow, sem).wait()
        pltpu.make_async_copy(rhtt_hbm.at[ridx.at[pl.ds(off, CH)]], rtrow, sem).wait()

    pltpu.sync_copy(s_hbm.at[pl.ds(base, BPW)], sidx)
    pltpu.sync_copy(r_hbm.at[pl.ds(base, BPW)], ridx)
    pltpu.sync_copy(o_hbm.at[pl.ds(base, BPW)], oidx)
    fire(0, sets[0])

    lane = lax.iota(jnp.int32, 16)

    def compute(c, bset):
        srow, orow, rrow, rtrow, _ = bset
        for g in range(NG):
            tvec = lane + g * L

            def dim_body(d, accs):
                accs = list(accs)
                for u in range(2):
                    dv = (lane + 2 * d + u) & 63
                    dv2 = dv + 64
                    s_e = plsc.load_gather(srow, [tvec, dv])
                    s_t = plsc.load_gather(srow, [tvec, dv2])
                    o_e = plsc.load_gather(orow, [tvec, dv])
                    o_t = plsc.load_gather(orow, [tvec, dv2])
                    r_e = plsc.load_gather(rrow, [tvec, dv])
                    r_h = plsc.load_gather(rtrow, [tvec, dv])
                    r_t = plsc.load_gather(rtrow, [tvec, dv2])
                    accs[3 * u + 0] = accs[3 * u + 0] + s_e * r_e * o_e
                    accs[3 * u + 1] = accs[3 * u + 1] + s_t * r_h
                    accs[3 * u + 2] = accs[3 * u + 2] + o_t * r_t
                return tuple(accs)

            z = jnp.zeros((L,), jnp.float32)
            acc6 = lax.fori_loop(0, D // 2, dim_body, (z,) * 6)
            b_acc = acc6[0] + acc6[3]
            h_acc = acc6[1] + acc6[4]
            t_acc = acc6[2] + acc6[5]
            res = (MULT
                   / (1.0 + jnp.exp(-b_acc))
                   / (1.0 + jnp.exp(-h_acc))
                   / (1.0 + jnp.exp(-t_acc)))
            outv[pl.ds(c * CH + g * L, L)] = res

    def pair_body(p, carry):
        for b in (0, 1):
            c = 2 * p + b

            @pl.when(c + 1 < NCHUNK)
            def _():
                fire(c + 1, sets[1 - b])

            drain(c, sets[b])
            compute(c, sets[b])
        return carry

    lax.fori_loop(0, NCHUNK // 2, pair_body, 0)
    pltpu.sync_copy(outv, out_hbm.at[pl.ds(base, BPW)])


def kernel(s, r, o, E, R, E_t, R_ht, R_tt):
    ee, rp, rhtt = _prep(E.T, E_t.T, R.T, R_ht.T, R_tt.T)
    return _sc_score(s, r, o, ee, rp, rhtt)
